# baseline (speedup 1.0000x reference)
import jax
import jax.numpy as jnp
from jax.experimental import pallas as pl
from jax.experimental.pallas import tpu as pltpu

_N = 100000
_D = 128
_BC = 5000
_S = _N // _BC
_K = 6


def _logic_kernel(x_hbm, c_hbm, wt_ref, b_ref, coef_ref, o_hbm,
                  xb, cb, ob, sx, sc, so):
    wt = wt_ref[...]
    bias = b_ref[...]
    a = coef_ref[0:1, :]
    m = coef_ref[1:2, :]

    def start_in(i, slot):
        rows = pl.ds(i * _BC, _BC)
        pltpu.make_async_copy(x_hbm.at[rows, :], xb.at[slot], sx.at[slot]).start()
        pltpu.make_async_copy(c_hbm.at[rows, :], cb.at[slot], sc.at[slot]).start()

    for j in range(_K - 1):
        start_in(j, j)

    def body(i, _):
        slot = jax.lax.rem(i, _K)

        @pl.when(i + _K - 1 < _S)
        def _():
            start_in(i + _K - 1, jax.lax.rem(i + _K - 1, _K))

        pltpu.make_async_copy(x_hbm.at[pl.ds(0, _BC), :], xb.at[slot], sx.at[slot]).wait()
        pltpu.make_async_copy(c_hbm.at[pl.ds(0, _BC), :], cb.at[slot], sc.at[slot]).wait()

        @pl.when(i >= _K)
        def _():
            pltpu.make_async_copy(ob.at[slot], o_hbm.at[pl.ds(0, _BC), :], so.at[slot]).wait()

        x = xb[slot]
        c = cb[slot]
        t = jnp.dot(x, wt, preferred_element_type=jnp.float32)
        t = jnp.maximum(t + bias, 0.0)
        ob[slot] = t + a * jnp.minimum(x, c) + m * jnp.maximum(x, c)

        rows = pl.ds(i * _BC, _BC)
        pltpu.make_async_copy(ob.at[slot], o_hbm.at[rows, :], so.at[slot]).start()
        return 0

    jax.lax.fori_loop(0, _S, body, 0)

    for j in range(_S - _K, _S):
        slot = j % _K
        pltpu.make_async_copy(ob.at[slot], o_hbm.at[pl.ds(0, _BC), :], so.at[slot]).wait()


def kernel(x, context, W, b, logical_weight, neural_weight):
    lw = jax.nn.sigmoid(logical_weight)
    nw = jax.nn.sigmoid(neural_weight)
    wt = (nw * W).T
    b2 = (nw * b).reshape(1, _D)
    coef = jnp.stack([
        jnp.full((_D,), (1.0 - nw) * lw, dtype=jnp.float32),
        jnp.full((_D,), (1.0 - nw) * (1.0 - lw), dtype=jnp.float32),
    ])
    return pl.pallas_call(
        _logic_kernel,
        in_specs=[
            pl.BlockSpec(memory_space=pltpu.MemorySpace.HBM),
            pl.BlockSpec(memory_space=pltpu.MemorySpace.HBM),
            pl.BlockSpec(memory_space=pltpu.MemorySpace.VMEM),
            pl.BlockSpec(memory_space=pltpu.MemorySpace.VMEM),
            pl.BlockSpec(memory_space=pltpu.MemorySpace.VMEM),
        ],
        out_specs=pl.BlockSpec(memory_space=pltpu.MemorySpace.HBM),
        out_shape=jax.ShapeDtypeStruct((_N, _D), jnp.float32),
        scratch_shapes=[
            pltpu.VMEM((_K, _BC, _D), jnp.float32),
            pltpu.VMEM((_K, _BC, _D), jnp.float32),
            pltpu.VMEM((_K, _BC, _D), jnp.float32),
            pltpu.SemaphoreType.DMA((_K,)),
            pltpu.SemaphoreType.DMA((_K,)),
            pltpu.SemaphoreType.DMA((_K,)),
        ],
    )(x, context, wt, b2, coef)


# speedup vs baseline: 1.0038x; 1.0038x over previous
import jax
import jax.numpy as jnp
from jax.experimental import pallas as pl
from jax.experimental.pallas import tpu as pltpu

_N = 100000
_D = 128
_BC = 2000
_S = _N // _BC
_K = 10


def _logic_kernel(x_hbm, c_hbm, wt_ref, b_ref, coef_ref, o_hbm,
                  xb, cb, ob, sx, sc, so):
    wt = wt_ref[...]
    bias = b_ref[...]
    a = coef_ref[0:1, :]
    m = coef_ref[1:2, :]

    def start_in(i, slot):
        rows = pl.ds(i * _BC, _BC)
        pltpu.make_async_copy(x_hbm.at[rows, :], xb.at[slot], sx.at[slot]).start()
        pltpu.make_async_copy(c_hbm.at[rows, :], cb.at[slot], sc.at[slot]).start()

    for j in range(_K - 1):
        start_in(j, j)

    def body(i, _):
        slot = jax.lax.rem(i, _K)

        @pl.when(i + _K - 1 < _S)
        def _():
            start_in(i + _K - 1, jax.lax.rem(i + _K - 1, _K))

        pltpu.make_async_copy(x_hbm.at[pl.ds(0, _BC), :], xb.at[slot], sx.at[slot]).wait()
        pltpu.make_async_copy(c_hbm.at[pl.ds(0, _BC), :], cb.at[slot], sc.at[slot]).wait()

        @pl.when(i >= _K)
        def _():
            pltpu.make_async_copy(ob.at[slot], o_hbm.at[pl.ds(0, _BC), :], so.at[slot]).wait()

        x = xb[slot]
        c = cb[slot]
        t = jnp.dot(x, wt, preferred_element_type=jnp.float32)
        t = jnp.maximum(t + bias, 0.0)
        ob[slot] = t + a * jnp.minimum(x, c) + m * jnp.maximum(x, c)

        rows = pl.ds(i * _BC, _BC)
        pltpu.make_async_copy(ob.at[slot], o_hbm.at[rows, :], so.at[slot]).start()
        return 0

    jax.lax.fori_loop(0, _S, body, 0)

    for j in range(_S - _K, _S):
        slot = j % _K
        pltpu.make_async_copy(ob.at[slot], o_hbm.at[pl.ds(0, _BC), :], so.at[slot]).wait()


def kernel(x, context, W, b, logical_weight, neural_weight):
    lw = jax.nn.sigmoid(logical_weight)
    nw = jax.nn.sigmoid(neural_weight)
    wt = (nw * W).T
    b2 = (nw * b).reshape(1, _D)
    coef = jnp.stack([
        jnp.full((_D,), (1.0 - nw) * lw, dtype=jnp.float32),
        jnp.full((_D,), (1.0 - nw) * (1.0 - lw), dtype=jnp.float32),
    ])
    return pl.pallas_call(
        _logic_kernel,
        in_specs=[
            pl.BlockSpec(memory_space=pltpu.MemorySpace.HBM),
            pl.BlockSpec(memory_space=pltpu.MemorySpace.HBM),
            pl.BlockSpec(memory_space=pltpu.MemorySpace.VMEM),
            pl.BlockSpec(memory_space=pltpu.MemorySpace.VMEM),
            pl.BlockSpec(memory_space=pltpu.MemorySpace.VMEM),
        ],
        out_specs=pl.BlockSpec(memory_space=pltpu.MemorySpace.HBM),
        out_shape=jax.ShapeDtypeStruct((_N, _D), jnp.float32),
        scratch_shapes=[
            pltpu.VMEM((_K, _BC, _D), jnp.float32),
            pltpu.VMEM((_K, _BC, _D), jnp.float32),
            pltpu.VMEM((_K, _BC, _D), jnp.float32),
            pltpu.SemaphoreType.DMA((_K,)),
            pltpu.SemaphoreType.DMA((_K,)),
            pltpu.SemaphoreType.DMA((_K,)),
        ],
    )(x, context, wt, b2, coef)


# speedup vs baseline: 1.0046x; 1.0008x over previous
import jax
import jax.numpy as jnp
from jax.experimental import pallas as pl
from jax.experimental.pallas import tpu as pltpu

_N = 100000
_D = 128
_BR = 16664


def _logic_kernel(x_ref, c_ref, wt_ref, b_ref, coef_ref, o_ref):
    x = x_ref[...]
    c = c_ref[...]
    t = jnp.dot(x, wt_ref[...], preferred_element_type=jnp.float32)
    t = jnp.maximum(t + b_ref[...], 0.0)
    a = coef_ref[0:1, :]
    m = coef_ref[1:2, :]
    o_ref[...] = t + a * jnp.minimum(x, c) + m * jnp.maximum(x, c)


def kernel(x, context, W, b, logical_weight, neural_weight):
    lw = jax.nn.sigmoid(logical_weight)
    nw = jax.nn.sigmoid(neural_weight)
    wt = (nw * W).T
    b2 = (nw * b).reshape(1, _D)
    coef = jnp.stack([
        jnp.full((_D,), (1.0 - nw) * lw, dtype=jnp.float32),
        jnp.full((_D,), (1.0 - nw) * (1.0 - lw), dtype=jnp.float32),
    ])
    grid = (_N + _BR - 1) // _BR
    return pl.pallas_call(
        _logic_kernel,
        grid=(grid,),
        in_specs=[
            pl.BlockSpec((_BR, _D), lambda i: (i, 0)),
            pl.BlockSpec((_BR, _D), lambda i: (i, 0)),
            pl.BlockSpec((_D, _D), lambda i: (0, 0)),
            pl.BlockSpec((1, _D), lambda i: (0, 0)),
            pl.BlockSpec((2, _D), lambda i: (0, 0)),
        ],
        out_specs=pl.BlockSpec((_BR, _D), lambda i: (i, 0)),
        out_shape=jax.ShapeDtypeStruct((_N, _D), jnp.float32),
        compiler_params=pltpu.CompilerParams(
            dimension_semantics=("parallel",),
        ),
    )(x, context, wt, b2, coef)


# speedup vs baseline: 1.0201x; 1.0154x over previous
import jax
import jax.numpy as jnp
from jax.experimental import pallas as pl
from jax.experimental.pallas import tpu as pltpu

_N = 100000
_D = 128
_BR = 16000


def _logic_kernel(x_ref, c_ref, wt_ref, b_ref, coef_ref, o_ref):
    x = x_ref[...]
    c = c_ref[...]
    t = jnp.dot(x, wt_ref[...], preferred_element_type=jnp.float32)
    t = jnp.maximum(t + b_ref[...], 0.0)
    a = coef_ref[0:1, :]
    m = coef_ref[1:2, :]
    o_ref[...] = t + a * jnp.minimum(x, c) + m * jnp.maximum(x, c)


def kernel(x, context, W, b, logical_weight, neural_weight):
    lw = jax.nn.sigmoid(logical_weight)
    nw = jax.nn.sigmoid(neural_weight)
    wt = (nw * W).T
    b2 = (nw * b).reshape(1, _D)
    coef = jnp.stack([
        jnp.full((_D,), (1.0 - nw) * lw, dtype=jnp.float32),
        jnp.full((_D,), (1.0 - nw) * (1.0 - lw), dtype=jnp.float32),
    ])
    grid = (_N + _BR - 1) // _BR
    return pl.pallas_call(
        _logic_kernel,
        grid=(grid,),
        in_specs=[
            pl.BlockSpec((_BR, _D), lambda i: (i, 0)),
            pl.BlockSpec((_BR, _D), lambda i: (i, 0)),
            pl.BlockSpec((_D, _D), lambda i: (0, 0)),
            pl.BlockSpec((1, _D), lambda i: (0, 0)),
            pl.BlockSpec((2, _D), lambda i: (0, 0)),
        ],
        out_specs=pl.BlockSpec((_BR, _D), lambda i: (i, 0)),
        out_shape=jax.ShapeDtypeStruct((_N, _D), jnp.float32),
        compiler_params=pltpu.CompilerParams(
            dimension_semantics=("arbitrary",),
        ),
    )(x, context, wt, b2, coef)


# speedup vs baseline: 1.0224x; 1.0023x over previous
import jax
import jax.numpy as jnp
from jax.experimental import pallas as pl
from jax.experimental.pallas import tpu as pltpu

_N = 100000
_D = 128
_BR = 16000


def _logic_kernel(x_ref, c_ref, wt_ref, b_ref, coef_ref, o_ref):
    x = x_ref[...]
    c = c_ref[...]
    t = jnp.dot(x, wt_ref[...], preferred_element_type=jnp.float32)
    t = jnp.maximum(t + b_ref[...], 0.0)
    a = coef_ref[0:1, :]
    m = coef_ref[1:2, :]
    o_ref[...] = t + a * jnp.minimum(x, c) + m * jnp.maximum(x, c)


def kernel(x, context, W, b, logical_weight, neural_weight):
    lw = jax.nn.sigmoid(logical_weight)
    nw = jax.nn.sigmoid(neural_weight)
    wt = (nw * W).T
    b2 = (nw * b).reshape(1, _D)
    coef = jnp.stack([
        jnp.full((_D,), (1.0 - nw) * lw, dtype=jnp.float32),
        jnp.full((_D,), (1.0 - nw) * (1.0 - lw), dtype=jnp.float32),
    ])
    grid = (_N + _BR - 1) // _BR
    return pl.pallas_call(
        _logic_kernel,
        grid=(grid,),
        in_specs=[
            pl.BlockSpec((_BR, _D), lambda i: (i, 0)),
            pl.BlockSpec((_BR, _D), lambda i: (i, 0)),
            pl.BlockSpec((_D, _D), lambda i: (0, 0)),
            pl.BlockSpec((1, _D), lambda i: (0, 0)),
            pl.BlockSpec((2, _D), lambda i: (0, 0)),
        ],
        out_specs=pl.BlockSpec((_BR, _D), lambda i: (i, 0)),
        out_shape=jax.ShapeDtypeStruct((_N, _D), jnp.float32),
        compiler_params=pltpu.CompilerParams(
            dimension_semantics=("parallel",),
        ),
    )(x, context, wt, b2, coef)
